# initial kernel scaffold (unmeasured)
import jax
import jax.numpy as jnp
from jax import lax
from jax.experimental import pallas as pl
from jax.experimental.pallas import tpu as pltpu


def kernel(
    x,
):
    def body(*refs):
        pass

    out_shape = jax.ShapeDtypeStruct(..., jnp.float32)
    return pl.pallas_call(body, out_shape=out_shape)(...)


# baseline (device time: 484313 ns/iter reference)
import jax
import jax.numpy as jnp
from jax import lax
from jax.experimental import pallas as pl
from jax.experimental.pallas import tpu as pltpu

M = 8192
N_OUT = 1024
CHUNK = 1024
N_CHUNKS = M // CHUNK


def kernel(x):
    x2 = x[0]
    my_x = lax.axis_index("x")
    send = lax.dynamic_slice(x2, (0, (1 - my_x) * N_OUT), (M, N_OUT))
    mine = lax.dynamic_slice(x2, (0, my_x * N_OUT), (M, N_OUT))

    def body(send_ref, mine_ref, out_ref, vmem_a, vmem_b, send_sem, recv_sem,
             copy_sem_a, copy_sem_b, copy_sem_o):
        mx = lax.axis_index("x")
        my = lax.axis_index("y")
        mz = lax.axis_index("z")
        rdma = pltpu.make_async_remote_copy(
            src_ref=send_ref,
            dst_ref=out_ref,
            send_sem=send_sem,
            recv_sem=recv_sem,
            device_id=(1 - mx, my, mz),
            device_id_type=pl.DeviceIdType.MESH,
        )
        rdma.start()
        rdma.wait()
        for c in range(N_CHUNKS):
            rows = pl.ds(c * CHUNK, CHUNK)
            cp_a = pltpu.make_async_copy(out_ref.at[rows], vmem_a, copy_sem_a)
            cp_b = pltpu.make_async_copy(mine_ref.at[rows], vmem_b, copy_sem_b)
            cp_a.start()
            cp_b.start()
            cp_a.wait()
            cp_b.wait()
            vmem_a[...] = vmem_a[...] + vmem_b[...]
            cp_o = pltpu.make_async_copy(vmem_a, out_ref.at[rows], copy_sem_o)
            cp_o.start()
            cp_o.wait()

    return pl.pallas_call(
        body,
        out_shape=jax.ShapeDtypeStruct((M, N_OUT), jnp.float32),
        in_specs=[
            pl.BlockSpec(memory_space=pl.ANY),
            pl.BlockSpec(memory_space=pl.ANY),
        ],
        out_specs=pl.BlockSpec(memory_space=pl.ANY),
        scratch_shapes=[
            pltpu.VMEM((CHUNK, N_OUT), jnp.float32),
            pltpu.VMEM((CHUNK, N_OUT), jnp.float32),
            pltpu.SemaphoreType.DMA,
            pltpu.SemaphoreType.DMA,
            pltpu.SemaphoreType.DMA,
            pltpu.SemaphoreType.DMA,
            pltpu.SemaphoreType.DMA,
        ],
    )(send, mine)


# device time: 394923 ns/iter; 1.2263x vs baseline; 1.2263x over previous
import jax
import jax.numpy as jnp
from jax import lax
from jax.experimental import pallas as pl
from jax.experimental.pallas import tpu as pltpu

M = 8192
N_OUT = 1024
CHUNK = 1024
N_CHUNKS = M // CHUNK


def kernel(x):
    def body(x_ref, out_ref, recv_bufs, mine_buf,
             send_sems, recv_sems, mine_sems, store_sems):
        mx = lax.axis_index("x")
        my = lax.axis_index("y")
        mz = lax.axis_index("z")
        pcol = (1 - mx) * N_OUT
        mycol = mx * N_OUT

        rdmas = []
        for c in range(N_CHUNKS):
            rows = pl.ds(c * CHUNK, CHUNK)
            rdma = pltpu.make_async_remote_copy(
                src_ref=x_ref.at[0, rows, pl.ds(pcol, N_OUT)],
                dst_ref=recv_bufs.at[c],
                send_sem=send_sems.at[c],
                recv_sem=recv_sems.at[c],
                device_id=(1 - mx, my, mz),
                device_id_type=pl.DeviceIdType.MESH,
            )
            rdma.start()
            rdmas.append(rdma)

        def mine_cp(c):
            return pltpu.make_async_copy(
                x_ref.at[0, pl.ds(c * CHUNK, CHUNK), pl.ds(mycol, N_OUT)],
                mine_buf.at[c % 2],
                mine_sems.at[c % 2],
            )

        mine_cp(0).start()
        store_cps = []
        for c in range(N_CHUNKS):
            if c + 1 < N_CHUNKS:
                mine_cp(c + 1).start()
            mine_cp(c).wait()
            rdmas[c].wait_recv()
            recv_bufs[c] = recv_bufs[c] + mine_buf[c % 2]
            st = pltpu.make_async_copy(
                recv_bufs.at[c],
                out_ref.at[pl.ds(c * CHUNK, CHUNK)],
                store_sems.at[c],
            )
            st.start()
            store_cps.append(st)

        for c in range(N_CHUNKS):
            rdmas[c].wait_send()
            store_cps[c].wait()

    return pl.pallas_call(
        body,
        out_shape=jax.ShapeDtypeStruct((M, N_OUT), jnp.float32),
        in_specs=[pl.BlockSpec(memory_space=pl.ANY)],
        out_specs=pl.BlockSpec(memory_space=pl.ANY),
        scratch_shapes=[
            pltpu.VMEM((N_CHUNKS, CHUNK, N_OUT), jnp.float32),
            pltpu.VMEM((2, CHUNK, N_OUT), jnp.float32),
            pltpu.SemaphoreType.DMA((N_CHUNKS,)),
            pltpu.SemaphoreType.DMA((N_CHUNKS,)),
            pltpu.SemaphoreType.DMA((2,)),
            pltpu.SemaphoreType.DMA((N_CHUNKS,)),
        ],
        compiler_params=pltpu.CompilerParams(
            vmem_limit_bytes=60 * 1024 * 1024,
        ),
    )(x)


# device time: 195304 ns/iter; 2.4798x vs baseline; 2.0221x over previous
import jax
import jax.numpy as jnp
from jax import lax
from jax.experimental import pallas as pl
from jax.experimental.pallas import tpu as pltpu

M = 8192
N_OUT = 1024
Q_ROWS = M // 4
SUB = 4
SROWS = Q_ROWS // SUB
F32 = jnp.float32


def kernel(x):
    def body(x_ref, out_ref, recv_x, p_buf, recv_y, recv_z, recv_d, mine_buf,
             send_x_sems, recv_x_sems, send_y2_sems, send_z2_sems,
             recv_y_sems, recv_z_sems, send_y3_sems, send_z3_sems,
             recv_d_sems, mine_sems, store_sems):
        mx = lax.axis_index("x")
        my = lax.axis_index("y")
        mz = lax.axis_index("z")
        pcol = (1 - mx) * N_OUT
        mycol = mx * N_OUT
        x_partner = (1 - mx, my, mz)
        y_nbr = (mx, 1 - my, mz)
        z_nbr = (mx, my, 1 - mz)
        q = 2 * my + mz
        qy = 2 * (1 - my) + mz
        qz = 2 * my + (1 - mz)
        qd = 2 * (1 - my) + (1 - mz)

        def srows(buf, s):
            return buf.at[pl.ds(s * SROWS, SROWS)]

        rdmas_x = []
        for s in range(SUB):
            rdma = pltpu.make_async_remote_copy(
                src_ref=x_ref.at[0, pl.ds(q * Q_ROWS + s * SROWS, SROWS),
                                 pl.ds(pcol, N_OUT)],
                dst_ref=srows(recv_x, s),
                send_sem=send_x_sems.at[s],
                recv_sem=recv_x_sems.at[s],
                device_id=x_partner,
                device_id_type=pl.DeviceIdType.MESH,
            )
            rdma.start()
            rdmas_x.append(rdma)

        def mine_cp(s):
            return pltpu.make_async_copy(
                x_ref.at[0, pl.ds(q * Q_ROWS + s * SROWS, SROWS),
                         pl.ds(mycol, N_OUT)],
                mine_buf.at[s % 2],
                mine_sems.at[s % 2],
            )

        def store(src, qi, s, sem_i):
            st = pltpu.make_async_copy(
                src, out_ref.at[pl.ds(qi * Q_ROWS + s * SROWS, SROWS)],
                store_sems.at[sem_i],
            )
            st.start()
            return st

        stores = []
        sends = []
        mine_cp(0).start()
        for s in range(SUB):
            if s + 1 < SUB:
                mine_cp(s + 1).start()
            mine_cp(s).wait()
            rdmas_x[s].wait_recv()
            p_buf[pl.ds(s * SROWS, SROWS)] = (
                recv_x[pl.ds(s * SROWS, SROWS)] + mine_buf[s % 2]
            )
            for dev, dst, ssem, rsem in (
                (y_nbr, recv_y, send_y2_sems, recv_y_sems),
                (z_nbr, recv_z, send_z2_sems, recv_z_sems),
            ):
                rd = pltpu.make_async_remote_copy(
                    src_ref=srows(p_buf, s),
                    dst_ref=srows(dst, s),
                    send_sem=ssem.at[s],
                    recv_sem=rsem.at[s],
                    device_id=dev,
                    device_id_type=pl.DeviceIdType.MESH,
                )
                rd.start()
                sends.append(rd)
            stores.append(store(srows(p_buf, s), q, s, s))

        rdmas_zr = []
        for s in range(SUB):
            rz = pltpu.make_async_remote_copy(
                src_ref=srows(p_buf, s), dst_ref=srows(recv_z, s),
                send_sem=send_z2_sems.at[s], recv_sem=recv_z_sems.at[s],
                device_id=z_nbr, device_id_type=pl.DeviceIdType.MESH,
            )
            rz.wait_recv()
            rdmas_zr.append(rz)
            if s < SUB // 2:
                rd = pltpu.make_async_remote_copy(
                    src_ref=srows(recv_z, s),
                    dst_ref=srows(recv_d, s),
                    send_sem=send_y3_sems.at[s],
                    recv_sem=recv_d_sems.at[s],
                    device_id=y_nbr,
                    device_id_type=pl.DeviceIdType.MESH,
                )
                rd.start()
                sends.append(rd)
            stores.append(store(srows(recv_z, s), qz, s, SUB + s))
        for s in range(SUB):
            ry = pltpu.make_async_remote_copy(
                src_ref=srows(p_buf, s), dst_ref=srows(recv_y, s),
                send_sem=send_y2_sems.at[s], recv_sem=recv_y_sems.at[s],
                device_id=y_nbr, device_id_type=pl.DeviceIdType.MESH,
            )
            ry.wait_recv()
            if s >= SUB // 2:
                rd = pltpu.make_async_remote_copy(
                    src_ref=srows(recv_y, s),
                    dst_ref=srows(recv_d, s),
                    send_sem=send_z3_sems.at[s - SUB // 2],
                    recv_sem=recv_d_sems.at[s],
                    device_id=z_nbr,
                    device_id_type=pl.DeviceIdType.MESH,
                )
                rd.start()
                sends.append(rd)
            stores.append(store(srows(recv_y, s), qy, s, 2 * SUB + s))
        for s in range(SUB):
            rdd = pltpu.make_async_remote_copy(
                src_ref=srows(recv_z, s), dst_ref=srows(recv_d, s),
                send_sem=send_y3_sems.at[s % (SUB // 2)],
                recv_sem=recv_d_sems.at[s],
                device_id=y_nbr, device_id_type=pl.DeviceIdType.MESH,
            )
            rdd.wait_recv()
            stores.append(store(srows(recv_d, s), qd, s, 3 * SUB + s))

        for rd in rdmas_x + sends:
            rd.wait_send()
        for st in stores:
            st.wait()

    return pl.pallas_call(
        body,
        out_shape=jax.ShapeDtypeStruct((M, N_OUT), F32),
        in_specs=[pl.BlockSpec(memory_space=pl.ANY)],
        out_specs=pl.BlockSpec(memory_space=pl.ANY),
        scratch_shapes=[
            pltpu.VMEM((Q_ROWS, N_OUT), F32),
            pltpu.VMEM((Q_ROWS, N_OUT), F32),
            pltpu.VMEM((Q_ROWS, N_OUT), F32),
            pltpu.VMEM((Q_ROWS, N_OUT), F32),
            pltpu.VMEM((Q_ROWS, N_OUT), F32),
            pltpu.VMEM((2, SROWS, N_OUT), F32),
            pltpu.SemaphoreType.DMA((SUB,)),
            pltpu.SemaphoreType.DMA((SUB,)),
            pltpu.SemaphoreType.DMA((SUB,)),
            pltpu.SemaphoreType.DMA((SUB,)),
            pltpu.SemaphoreType.DMA((SUB,)),
            pltpu.SemaphoreType.DMA((SUB,)),
            pltpu.SemaphoreType.DMA((SUB // 2,)),
            pltpu.SemaphoreType.DMA((SUB // 2,)),
            pltpu.SemaphoreType.DMA((SUB,)),
            pltpu.SemaphoreType.DMA((2,)),
            pltpu.SemaphoreType.DMA((4 * SUB,)),
        ],
        compiler_params=pltpu.CompilerParams(
            vmem_limit_bytes=60 * 1024 * 1024,
        ),
    )(x)


# device time: 194569 ns/iter; 2.4892x vs baseline; 1.0038x over previous
import jax
import jax.numpy as jnp
from jax import lax
from jax.experimental import pallas as pl
from jax.experimental.pallas import tpu as pltpu

M = 8192
N_OUT = 1024
Q_ROWS = M // 4
SUB = 8
SROWS = Q_ROWS // SUB
DK = 3
KY = 2
KZ = SUB - DK - KY
F32 = jnp.float32


def kernel(x):
    def body(x_ref, out_ref, recv_x, recv_y, recv_z, recv_d, mine_buf,
             send_x_sems, recv_x_sems, send_y2_sems, send_z2_sems,
             recv_y_sems, recv_z_sems, send_y3_sems, send_z3_sems,
             recv_d_sems, mine_sems, store_sems):
        mx = lax.axis_index("x")
        my = lax.axis_index("y")
        mz = lax.axis_index("z")
        pcol = (1 - mx) * N_OUT
        mycol = mx * N_OUT
        x_partner = (1 - mx, my, mz)
        y_nbr = (mx, 1 - my, mz)
        z_nbr = (mx, my, 1 - mz)
        q = 2 * my + mz
        qy = 2 * (1 - my) + mz
        qz = 2 * my + (1 - mz)
        qd = 2 * (1 - my) + (1 - mz)

        def srows(buf, s):
            return buf.at[pl.ds(s * SROWS, SROWS)]

        def x_src(qi, s):
            return x_ref.at[0, pl.ds(qi * Q_ROWS + s * SROWS, SROWS),
                            pl.ds(pcol, N_OUT)]

        def piece(i):
            return (q, i) if i < SUB else (qd, i - SUB)

        rdmas_x = []
        for i in range(SUB + DK):
            qi, s = piece(i)
            rdma = pltpu.make_async_remote_copy(
                src_ref=x_src(qi, s),
                dst_ref=srows(recv_x, i),
                send_sem=send_x_sems.at[i],
                recv_sem=recv_x_sems.at[i],
                device_id=x_partner,
                device_id_type=pl.DeviceIdType.MESH,
            )
            rdma.start()
            rdmas_x.append(rdma)

        def mine_cp(i):
            qi, s = piece(i)
            return pltpu.make_async_copy(
                x_ref.at[0, pl.ds(qi * Q_ROWS + s * SROWS, SROWS),
                         pl.ds(mycol, N_OUT)],
                mine_buf.at[i % 2],
                mine_sems.at[i % 2],
            )

        def store(src, qi, s, sem_i):
            st = pltpu.make_async_copy(
                src, out_ref.at[pl.ds(qi * Q_ROWS + s * SROWS, SROWS)],
                store_sems.at[sem_i],
            )
            st.start()
            return st

        stores = []
        sends = []
        mine_cp(0).start()
        for i in range(SUB + DK):
            qi, s = piece(i)
            if i + 1 < SUB + DK:
                mine_cp(i + 1).start()
            mine_cp(i).wait()
            rdmas_x[i].wait_recv()
            recv_x[pl.ds(i * SROWS, SROWS)] = (
                recv_x[pl.ds(i * SROWS, SROWS)] + mine_buf[i % 2]
            )
            if i < SUB:
                for dev, dst, ssem, rsem in (
                    (y_nbr, recv_y, send_y2_sems, recv_y_sems),
                    (z_nbr, recv_z, send_z2_sems, recv_z_sems),
                ):
                    rd = pltpu.make_async_remote_copy(
                        src_ref=srows(recv_x, i),
                        dst_ref=srows(dst, i),
                        send_sem=ssem.at[i],
                        recv_sem=rsem.at[i],
                        device_id=dev,
                        device_id_type=pl.DeviceIdType.MESH,
                    )
                    rd.start()
                    sends.append(rd)
            stores.append(store(srows(recv_x, i), qi, s, i))

        for s in range(SUB):
            rz = pltpu.make_async_remote_copy(
                src_ref=srows(recv_x, s), dst_ref=srows(recv_z, s),
                send_sem=send_z2_sems.at[s], recv_sem=recv_z_sems.at[s],
                device_id=z_nbr, device_id_type=pl.DeviceIdType.MESH,
            )
            rz.wait_recv()
            if DK <= s < DK + KY:
                rd = pltpu.make_async_remote_copy(
                    src_ref=srows(recv_z, s),
                    dst_ref=srows(recv_d, s - DK),
                    send_sem=send_y3_sems.at[s - DK],
                    recv_sem=recv_d_sems.at[s - DK],
                    device_id=y_nbr,
                    device_id_type=pl.DeviceIdType.MESH,
                )
                rd.start()
                sends.append(rd)
            stores.append(store(srows(recv_z, s), qz, s, (SUB + DK) + s))
        for s in range(SUB):
            ry = pltpu.make_async_remote_copy(
                src_ref=srows(recv_x, s), dst_ref=srows(recv_y, s),
                send_sem=send_y2_sems.at[s], recv_sem=recv_y_sems.at[s],
                device_id=y_nbr, device_id_type=pl.DeviceIdType.MESH,
            )
            ry.wait_recv()
            if s >= DK + KY:
                rd = pltpu.make_async_remote_copy(
                    src_ref=srows(recv_y, s),
                    dst_ref=srows(recv_d, s - DK),
                    send_sem=send_z3_sems.at[s - DK - KY],
                    recv_sem=recv_d_sems.at[s - DK],
                    device_id=z_nbr,
                    device_id_type=pl.DeviceIdType.MESH,
                )
                rd.start()
                sends.append(rd)
            stores.append(store(srows(recv_y, s), qy, s, (SUB + DK) + SUB + s))
        for j in range(KY + KZ):
            rdd = pltpu.make_async_remote_copy(
                src_ref=srows(recv_x, 0), dst_ref=srows(recv_d, j),
                send_sem=send_x_sems.at[0],
                recv_sem=recv_d_sems.at[j],
                device_id=y_nbr, device_id_type=pl.DeviceIdType.MESH,
            )
            rdd.wait_recv()
            stores.append(
                store(srows(recv_d, j), qd, DK + j,
                      (SUB + DK) + 2 * SUB + j)
            )

        for rd in rdmas_x + sends:
            rd.wait_send()
        for st in stores:
            st.wait()

    n_stores = (SUB + DK) + 2 * SUB + (KY + KZ)
    return pl.pallas_call(
        body,
        out_shape=jax.ShapeDtypeStruct((M, N_OUT), F32),
        in_specs=[pl.BlockSpec(memory_space=pl.ANY)],
        out_specs=pl.BlockSpec(memory_space=pl.ANY),
        scratch_shapes=[
            pltpu.VMEM(((SUB + DK) * SROWS, N_OUT), F32),
            pltpu.VMEM((Q_ROWS, N_OUT), F32),
            pltpu.VMEM((Q_ROWS, N_OUT), F32),
            pltpu.VMEM(((KY + KZ) * SROWS, N_OUT), F32),
            pltpu.VMEM((2, SROWS, N_OUT), F32),
            pltpu.SemaphoreType.DMA((SUB + DK,)),
            pltpu.SemaphoreType.DMA((SUB + DK,)),
            pltpu.SemaphoreType.DMA((SUB,)),
            pltpu.SemaphoreType.DMA((SUB,)),
            pltpu.SemaphoreType.DMA((SUB,)),
            pltpu.SemaphoreType.DMA((SUB,)),
            pltpu.SemaphoreType.DMA((KY,)),
            pltpu.SemaphoreType.DMA((KZ,)),
            pltpu.SemaphoreType.DMA((KY + KZ,)),
            pltpu.SemaphoreType.DMA((2,)),
            pltpu.SemaphoreType.DMA((n_stores,)),
        ],
        compiler_params=pltpu.CompilerParams(
            vmem_limit_bytes=60 * 1024 * 1024,
        ),
    )(x)


# device time: 173775 ns/iter; 2.7870x vs baseline; 1.1197x over previous
import jax
import jax.numpy as jnp
from jax import lax
from jax.experimental import pallas as pl
from jax.experimental.pallas import tpu as pltpu

M = 8192
N_OUT = 1024
Q_ROWS = M // 4
SUB = 8
SROWS = Q_ROWS // SUB
DK = 3
KY = 2
KZ = SUB - DK - KY
F32 = jnp.float32


def kernel(x):
    def body(x_ref, out_ref, recv_x, recv_y, recv_z, recv_d, mine_buf,
             send_x_sems, recv_x_sems, send_y2_sems, send_z2_sems,
             recv_y_sems, recv_z_sems, send_y3_sems, send_z3_sems,
             recv_d_sems, mine_sems, store_sems):
        mx = lax.axis_index("x")
        my = lax.axis_index("y")
        mz = lax.axis_index("z")
        pcol = (1 - mx) * N_OUT
        mycol = mx * N_OUT
        x_partner = (1 - mx, my, mz)
        y_nbr = (mx, 1 - my, mz)
        z_nbr = (mx, my, 1 - mz)
        q = 2 * my + mz
        qy = 2 * (1 - my) + mz
        qz = 2 * my + (1 - mz)
        qd = 2 * (1 - my) + (1 - mz)

        def srows(buf, s):
            return buf.at[pl.ds(s * SROWS, SROWS)]

        def x_src(qi, s):
            return x_ref.at[0, pl.ds(qi * Q_ROWS + s * SROWS, SROWS),
                            pl.ds(pcol, N_OUT)]

        def piece(i):
            return (q, i) if i < SUB else (qd, i - SUB)

        rdmas_x = []
        for i in range(SUB + DK):
            qi, s = piece(i)
            rdma = pltpu.make_async_remote_copy(
                src_ref=x_src(qi, s),
                dst_ref=srows(recv_x, i),
                send_sem=send_x_sems.at[i],
                recv_sem=recv_x_sems.at[i],
                device_id=x_partner,
                device_id_type=pl.DeviceIdType.MESH,
            )
            rdma.start()
            rdmas_x.append(rdma)

        def mine_cp(i):
            qi, s = piece(i)
            return pltpu.make_async_copy(
                x_ref.at[0, pl.ds(qi * Q_ROWS + s * SROWS, SROWS),
                         pl.ds(mycol, N_OUT)],
                mine_buf.at[i % 2],
                mine_sems.at[i % 2],
            )

        def store(src, qi, s, sem_i):
            st = pltpu.make_async_copy(
                src, out_ref.at[pl.ds(qi * Q_ROWS + s * SROWS, SROWS)],
                store_sems.at[sem_i],
            )
            st.start()
            return st

        stores = []
        sends = []
        sem_ctr = [0]

        def next_sem():
            sem_ctr[0] += 1
            return sem_ctr[0] - 1

        def wait_recv_of(dst_slice, rsem):
            pltpu.make_async_remote_copy(
                src_ref=srows(recv_x, 0), dst_ref=dst_slice,
                send_sem=send_x_sems.at[0], recv_sem=rsem,
                device_id=x_partner, device_id_type=pl.DeviceIdType.MESH,
            ).wait_recv()

        def fwd_z(s):
            wait_recv_of(srows(recv_z, s), recv_z_sems.at[s])
            rd = pltpu.make_async_remote_copy(
                src_ref=srows(recv_z, s), dst_ref=srows(recv_d, s - DK),
                send_sem=send_y3_sems.at[s - DK],
                recv_sem=recv_d_sems.at[s - DK],
                device_id=y_nbr, device_id_type=pl.DeviceIdType.MESH,
            )
            rd.start()
            sends.append(rd)
            stores.append(store(srows(recv_z, s), qz, s, next_sem()))

        def fwd_y(s):
            wait_recv_of(srows(recv_y, s), recv_y_sems.at[s])
            rd = pltpu.make_async_remote_copy(
                src_ref=srows(recv_y, s), dst_ref=srows(recv_d, s - DK),
                send_sem=send_z3_sems.at[s - DK - KY],
                recv_sem=recv_d_sems.at[s - DK],
                device_id=z_nbr, device_id_type=pl.DeviceIdType.MESH,
            )
            rd.start()
            sends.append(rd)
            stores.append(store(srows(recv_y, s), qy, s, next_sem()))

        post_ops = {
            5: [(fwd_z, DK)],
            6: [(fwd_z, DK + 1)],
            7: [(fwd_y, DK + KY)],
            8: [(fwd_y, DK + KY + 1)],
            9: [(fwd_y, DK + KY + 2)],
        }

        mine_cp(0).start()
        for i in range(SUB + DK):
            qi, s = piece(i)
            if i + 1 < SUB + DK:
                mine_cp(i + 1).start()
            mine_cp(i).wait()
            rdmas_x[i].wait_recv()
            recv_x[pl.ds(i * SROWS, SROWS)] = (
                recv_x[pl.ds(i * SROWS, SROWS)] + mine_buf[i % 2]
            )
            if i < SUB:
                for dev, dst, ssem, rsem in (
                    (y_nbr, recv_y, send_y2_sems, recv_y_sems),
                    (z_nbr, recv_z, send_z2_sems, recv_z_sems),
                ):
                    rd = pltpu.make_async_remote_copy(
                        src_ref=srows(recv_x, i),
                        dst_ref=srows(dst, i),
                        send_sem=ssem.at[i],
                        recv_sem=rsem.at[i],
                        device_id=dev,
                        device_id_type=pl.DeviceIdType.MESH,
                    )
                    rd.start()
                    sends.append(rd)
            stores.append(store(srows(recv_x, i), qi, s, next_sem()))
            for fn, fs in post_ops.get(i, ()):
                fn(fs)

        for s in range(SUB):
            if not (DK <= s < DK + KY):
                wait_recv_of(srows(recv_z, s), recv_z_sems.at[s])
                stores.append(store(srows(recv_z, s), qz, s, next_sem()))
        for s in range(SUB):
            if not (s >= DK + KY):
                wait_recv_of(srows(recv_y, s), recv_y_sems.at[s])
                stores.append(store(srows(recv_y, s), qy, s, next_sem()))
        for j in range(KY + KZ):
            wait_recv_of(srows(recv_d, j), recv_d_sems.at[j])
            stores.append(store(srows(recv_d, j), qd, DK + j, next_sem()))

        for rd in rdmas_x + sends:
            rd.wait_send()
        for st in stores:
            st.wait()

    n_stores = (SUB + DK) + 2 * SUB + (KY + KZ)
    return pl.pallas_call(
        body,
        out_shape=jax.ShapeDtypeStruct((M, N_OUT), F32),
        in_specs=[pl.BlockSpec(memory_space=pl.ANY)],
        out_specs=pl.BlockSpec(memory_space=pl.ANY),
        scratch_shapes=[
            pltpu.VMEM(((SUB + DK) * SROWS, N_OUT), F32),
            pltpu.VMEM((Q_ROWS, N_OUT), F32),
            pltpu.VMEM((Q_ROWS, N_OUT), F32),
            pltpu.VMEM(((KY + KZ) * SROWS, N_OUT), F32),
            pltpu.VMEM((2, SROWS, N_OUT), F32),
            pltpu.SemaphoreType.DMA((SUB + DK,)),
            pltpu.SemaphoreType.DMA((SUB + DK,)),
            pltpu.SemaphoreType.DMA((SUB,)),
            pltpu.SemaphoreType.DMA((SUB,)),
            pltpu.SemaphoreType.DMA((SUB,)),
            pltpu.SemaphoreType.DMA((SUB,)),
            pltpu.SemaphoreType.DMA((KY,)),
            pltpu.SemaphoreType.DMA((KZ,)),
            pltpu.SemaphoreType.DMA((KY + KZ,)),
            pltpu.SemaphoreType.DMA((2,)),
            pltpu.SemaphoreType.DMA((n_stores,)),
        ],
        compiler_params=pltpu.CompilerParams(
            vmem_limit_bytes=60 * 1024 * 1024,
        ),
    )(x)


# device time: 173388 ns/iter; 2.7932x vs baseline; 1.0022x over previous
import jax
import jax.numpy as jnp
from jax import lax
from jax.experimental import pallas as pl
from jax.experimental.pallas import tpu as pltpu

M = 8192
N_OUT = 1024
Q_ROWS = M // 4
SUB = 8
SROWS = Q_ROWS // SUB
DK = 3
KY = 2
KZ = SUB - DK - KY
F32 = jnp.float32


def kernel(x):
    def body(x_ref, out_ref, recv_x, recv_y, recv_z, recv_d, mine_buf,
             send_x_sems, recv_x_sems, send_y2_sems, send_z2_sems,
             recv_y_sems, recv_z_sems, send_y3_sems, send_z3_sems,
             recv_d_sems, mine_sems, store_sems):
        mx = lax.axis_index("x")
        my = lax.axis_index("y")
        mz = lax.axis_index("z")
        pcol = (1 - mx) * N_OUT
        mycol = mx * N_OUT
        x_partner = (1 - mx, my, mz)
        y_nbr = (mx, 1 - my, mz)
        z_nbr = (mx, my, 1 - mz)
        q = 2 * my + mz
        qy = 2 * (1 - my) + mz
        qz = 2 * my + (1 - mz)
        qd = 2 * (1 - my) + (1 - mz)

        def srows(buf, s):
            return buf.at[pl.ds(s * SROWS, SROWS)]

        def x_src(qi, s):
            return x_ref.at[0, pl.ds(qi * Q_ROWS + s * SROWS, SROWS),
                            pl.ds(pcol, N_OUT)]

        def piece(i):
            return (q, i) if i < SUB else (qd, i - SUB)

        rdmas_x = []
        for i in range(SUB + DK):
            qi, s = piece(i)
            rdma = pltpu.make_async_remote_copy(
                src_ref=x_src(qi, s),
                dst_ref=srows(recv_x, i),
                send_sem=send_x_sems.at[i],
                recv_sem=recv_x_sems.at[i],
                device_id=x_partner,
                device_id_type=pl.DeviceIdType.MESH,
            )
            rdma.start()
            rdmas_x.append(rdma)

        def mine_cp(i):
            qi, s = piece(i)
            return pltpu.make_async_copy(
                x_ref.at[0, pl.ds(qi * Q_ROWS + s * SROWS, SROWS),
                         pl.ds(mycol, N_OUT)],
                mine_buf.at[i % 2],
                mine_sems.at[i % 2],
            )

        def store(src, qi, s, sem_i):
            st = pltpu.make_async_copy(
                src, out_ref.at[pl.ds(qi * Q_ROWS + s * SROWS, SROWS)],
                store_sems.at[sem_i],
            )
            st.start()
            return st

        stores = []
        sends = []
        sem_ctr = [0]

        def next_sem():
            sem_ctr[0] += 1
            return sem_ctr[0] - 1

        def wait_recv_of(dst_slice, rsem):
            pltpu.make_async_remote_copy(
                src_ref=srows(recv_x, 0), dst_ref=dst_slice,
                send_sem=send_x_sems.at[0], recv_sem=rsem,
                device_id=x_partner, device_id_type=pl.DeviceIdType.MESH,
            ).wait_recv()

        def fwd_z(s):
            wait_recv_of(srows(recv_z, s), recv_z_sems.at[s])
            rd = pltpu.make_async_remote_copy(
                src_ref=srows(recv_z, s), dst_ref=srows(recv_d, s - DK),
                send_sem=send_y3_sems.at[s - DK],
                recv_sem=recv_d_sems.at[s - DK],
                device_id=y_nbr, device_id_type=pl.DeviceIdType.MESH,
            )
            rd.start()
            sends.append(rd)
            stores.append(store(srows(recv_z, s), qz, s, next_sem()))

        def fwd_y(s):
            wait_recv_of(srows(recv_y, s), recv_y_sems.at[s])
            rd = pltpu.make_async_remote_copy(
                src_ref=srows(recv_y, s), dst_ref=srows(recv_d, s - DK),
                send_sem=send_z3_sems.at[s - DK - KY],
                recv_sem=recv_d_sems.at[s - DK],
                device_id=z_nbr, device_id_type=pl.DeviceIdType.MESH,
            )
            rd.start()
            sends.append(rd)
            stores.append(store(srows(recv_y, s), qy, s, next_sem()))

        post_ops = {
            4: [(fwd_z, DK)],
            5: [(fwd_z, DK + 1)],
            6: [(fwd_y, DK + KY)],
            7: [(fwd_y, DK + KY + 1)],
            8: [(fwd_y, DK + KY + 2)],
        }

        mine_cp(0).start()
        for i in range(SUB + DK):
            qi, s = piece(i)
            if i + 1 < SUB + DK:
                mine_cp(i + 1).start()
            mine_cp(i).wait()
            rdmas_x[i].wait_recv()
            recv_x[pl.ds(i * SROWS, SROWS)] = (
                recv_x[pl.ds(i * SROWS, SROWS)] + mine_buf[i % 2]
            )
            if i < SUB:
                for dev, dst, ssem, rsem in (
                    (y_nbr, recv_y, send_y2_sems, recv_y_sems),
                    (z_nbr, recv_z, send_z2_sems, recv_z_sems),
                ):
                    rd = pltpu.make_async_remote_copy(
                        src_ref=srows(recv_x, i),
                        dst_ref=srows(dst, i),
                        send_sem=ssem.at[i],
                        recv_sem=rsem.at[i],
                        device_id=dev,
                        device_id_type=pl.DeviceIdType.MESH,
                    )
                    rd.start()
                    sends.append(rd)
            stores.append(store(srows(recv_x, i), qi, s, next_sem()))
            for fn, fs in post_ops.get(i, ()):
                fn(fs)

        for s in range(SUB):
            if not (DK <= s < DK + KY):
                wait_recv_of(srows(recv_z, s), recv_z_sems.at[s])
                stores.append(store(srows(recv_z, s), qz, s, next_sem()))
        for s in range(SUB):
            if not (s >= DK + KY):
                wait_recv_of(srows(recv_y, s), recv_y_sems.at[s])
                stores.append(store(srows(recv_y, s), qy, s, next_sem()))
        for j in range(KY + KZ):
            wait_recv_of(srows(recv_d, j), recv_d_sems.at[j])
            stores.append(store(srows(recv_d, j), qd, DK + j, next_sem()))

        for rd in rdmas_x + sends:
            rd.wait_send()
        for st in stores:
            st.wait()

    n_stores = (SUB + DK) + 2 * SUB + (KY + KZ)
    return pl.pallas_call(
        body,
        out_shape=jax.ShapeDtypeStruct((M, N_OUT), F32),
        in_specs=[pl.BlockSpec(memory_space=pl.ANY)],
        out_specs=pl.BlockSpec(memory_space=pl.ANY),
        scratch_shapes=[
            pltpu.VMEM(((SUB + DK) * SROWS, N_OUT), F32),
            pltpu.VMEM((Q_ROWS, N_OUT), F32),
            pltpu.VMEM((Q_ROWS, N_OUT), F32),
            pltpu.VMEM(((KY + KZ) * SROWS, N_OUT), F32),
            pltpu.VMEM((2, SROWS, N_OUT), F32),
            pltpu.SemaphoreType.DMA((SUB + DK,)),
            pltpu.SemaphoreType.DMA((SUB + DK,)),
            pltpu.SemaphoreType.DMA((SUB,)),
            pltpu.SemaphoreType.DMA((SUB,)),
            pltpu.SemaphoreType.DMA((SUB,)),
            pltpu.SemaphoreType.DMA((SUB,)),
            pltpu.SemaphoreType.DMA((KY,)),
            pltpu.SemaphoreType.DMA((KZ,)),
            pltpu.SemaphoreType.DMA((KY + KZ,)),
            pltpu.SemaphoreType.DMA((2,)),
            pltpu.SemaphoreType.DMA((n_stores,)),
        ],
        compiler_params=pltpu.CompilerParams(
            vmem_limit_bytes=60 * 1024 * 1024,
        ),
    )(x)
